# baseline (speedup 1.0000x reference)
import jax
import jax.numpy as jnp
from jax.experimental import pallas as pl
from jax.experimental.pallas import tpu as pltpu

_THRESHOLD = 5.0
_GAMMA = 0.1
_LAMBDA = 0.1
_EPS = 1e-6
_BLOCK = 3200


def _gcpl_kernel(label_ref, protos_ref, protos_l_ref, counts_ref, feat_ref,
                 out_ref, acc_ref):
    i = pl.program_id(0)
    nb = pl.num_programs(0)

    @pl.when(i == 0)
    def _():
        acc_ref[0] = 0.0

    fb = feat_ref[:]
    g = fb - _EPS
    x = protos_ref[...]
    diff = x - g[None]
    dist2 = jnp.sum(diff * diff, axis=2)
    w = jnp.exp(-_GAMMA * dist2)
    acc_ref[0] += jnp.sum(w)

    @pl.when(i == nb - 1)
    def _():
        d_dim = fb.shape[1]
        pls = protos_l_ref[0]
        diffl = pls - fb + _EPS
        d2l = jnp.sum(diffl * diffl, axis=1, keepdims=True)
        dl = jnp.sqrt(d2l)
        min_d = jnp.min(dl)
        n_p = pls.shape[0]
        row_iota = jax.lax.broadcasted_iota(jnp.int32, (n_p, 1), 0)
        idx = jnp.min(jnp.where(dl == min_d, row_iota, n_p))
        lbl = label_ref[0]
        cnt = counts_ref[...].astype(jnp.float32)
        crow_iota = jax.lax.broadcasted_iota(jnp.int32, cnt.shape, 0)
        ccol_iota = jax.lax.broadcasted_iota(jnp.int32, cnt.shape, 1)
        c = jnp.sum(jnp.where(
            (crow_iota == idx) & (ccol_iota == lbl), cnt, 0.0))
        proto_i = jnp.sum(jnp.where(row_iota == idx, pls, 0.0),
                          axis=0, keepdims=True)
        updated = (proto_i * c + fb) / (c + 1.0)
        take = min_d < _THRESHOLD
        closest = jnp.where(take, updated, fb)
        p_loss = jnp.sum((fb - closest + _EPS) ** 2)
        d_upd = jnp.sqrt(jnp.sum((updated - fb + _EPS) ** 2))
        w_new = jnp.exp(-_GAMMA * (d_upd * d_upd))
        w_old = jnp.exp(-_GAMMA * (min_d * min_d))
        delta = jnp.where(take, w_new - w_old, 0.0)
        append_w = jnp.where(take, 0.0,
                             jnp.exp(-_GAMMA * (d_dim * _EPS * _EPS)))
        s_label = jnp.sum(jnp.exp(-_GAMMA * (dl * dl)))
        one = acc_ref[0] + delta + append_w
        num = s_label + delta + append_w
        prob = jnp.where(one > 0.0, num / one, one + 0.1)
        prob = jnp.where(prob > 0.0, prob, prob + 1e-6)
        loss = -jnp.log(prob) + _LAMBDA * p_loss
        out_ref[...] = jnp.full((1, 1), loss, dtype=jnp.float32)


def kernel(feature, label, prototypes, sample_counts):
    L, P, D = prototypes.shape
    label_arr = jnp.asarray(label, jnp.int32).reshape(1)
    nb = (L * P) // _BLOCK
    grid_spec = pltpu.PrefetchScalarGridSpec(
        num_scalar_prefetch=1,
        grid=(nb,),
        in_specs=[
            pl.BlockSpec((_BLOCK // 16, 16, D), lambda i, lbl: (i, 0, 0)),
            pl.BlockSpec((1, P, D), lambda i, lbl: (lbl[0], 0, 0)),
            pl.BlockSpec((P, L), lambda i, lbl: (0, 0)),
            pl.BlockSpec((1, D), lambda i, lbl: (0, 0)),
        ],
        out_specs=pl.BlockSpec((1, 1), lambda i, lbl: (0, 0)),
        scratch_shapes=[pltpu.SMEM((1,), jnp.float32)],
    )
    out = pl.pallas_call(
        _gcpl_kernel,
        grid_spec=grid_spec,
        out_shape=jax.ShapeDtypeStruct((1, 1), jnp.float32),
        compiler_params=pltpu.CompilerParams(
            dimension_semantics=("arbitrary",)),
    )(label_arr, prototypes, prototypes, sample_counts.T, feature)
    return out[0, 0]


# speedup vs baseline: 1.0176x; 1.0176x over previous
import jax
import jax.numpy as jnp
from jax.experimental import pallas as pl
from jax.experimental.pallas import tpu as pltpu

_THRESHOLD = 5.0
_GAMMA = 0.1
_LAMBDA = 0.1
_EPS = 1e-6
_BLOCK = 4000


def _gcpl_kernel(label_ref, protos_ref, protos_l_ref, counts_ref, feat_ref,
                 out_ref, acc_ref):
    i = pl.program_id(0)
    nb = pl.num_programs(0)

    @pl.when(i == 0)
    def _():
        acc_ref[0] = 0.0

    fb = feat_ref[:]
    g = fb - _EPS
    x = protos_ref[...]
    diff = x - g[None]
    dist2 = jnp.sum(diff * diff, axis=2)
    w = jnp.exp(-_GAMMA * dist2)
    acc_ref[0] += jnp.sum(w)

    @pl.when(i == nb - 1)
    def _():
        d_dim = fb.shape[1]
        pls = protos_l_ref[0]
        diffl = pls - fb + _EPS
        d2l = jnp.sum(diffl * diffl, axis=1, keepdims=True)
        dl = jnp.sqrt(d2l)
        min_d = jnp.min(dl)
        n_p = pls.shape[0]
        row_iota = jax.lax.broadcasted_iota(jnp.int32, (n_p, 1), 0)
        idx = jnp.min(jnp.where(dl == min_d, row_iota, n_p))
        lbl = label_ref[0]
        cnt = counts_ref[...].astype(jnp.float32)
        crow_iota = jax.lax.broadcasted_iota(jnp.int32, cnt.shape, 0)
        ccol_iota = jax.lax.broadcasted_iota(jnp.int32, cnt.shape, 1)
        c = jnp.sum(jnp.where(
            (crow_iota == idx) & (ccol_iota == lbl), cnt, 0.0))
        proto_i = jnp.sum(jnp.where(row_iota == idx, pls, 0.0),
                          axis=0, keepdims=True)
        updated = (proto_i * c + fb) / (c + 1.0)
        take = min_d < _THRESHOLD
        closest = jnp.where(take, updated, fb)
        p_loss = jnp.sum((fb - closest + _EPS) ** 2)
        d_upd = jnp.sqrt(jnp.sum((updated - fb + _EPS) ** 2))
        w_new = jnp.exp(-_GAMMA * (d_upd * d_upd))
        w_old = jnp.exp(-_GAMMA * (min_d * min_d))
        delta = jnp.where(take, w_new - w_old, 0.0)
        append_w = jnp.where(take, 0.0,
                             jnp.exp(-_GAMMA * (d_dim * _EPS * _EPS)))
        s_label = jnp.sum(jnp.exp(-_GAMMA * (dl * dl)))
        one = acc_ref[0] + delta + append_w
        num = s_label + delta + append_w
        prob = jnp.where(one > 0.0, num / one, one + 0.1)
        prob = jnp.where(prob > 0.0, prob, prob + 1e-6)
        loss = -jnp.log(prob) + _LAMBDA * p_loss
        out_ref[...] = jnp.full((1, 1), loss, dtype=jnp.float32)


def kernel(feature, label, prototypes, sample_counts):
    L, P, D = prototypes.shape
    label_arr = jnp.asarray(label, jnp.int32).reshape(1)
    nb = (L * P) // _BLOCK
    grid_spec = pltpu.PrefetchScalarGridSpec(
        num_scalar_prefetch=1,
        grid=(nb,),
        in_specs=[
            pl.BlockSpec((_BLOCK // 16, 16, D), lambda i, lbl: (i, 0, 0)),
            pl.BlockSpec((1, P, D), lambda i, lbl: (lbl[0], 0, 0)),
            pl.BlockSpec((P, L), lambda i, lbl: (0, 0)),
            pl.BlockSpec((1, D), lambda i, lbl: (0, 0)),
        ],
        out_specs=pl.BlockSpec((1, 1), lambda i, lbl: (0, 0)),
        scratch_shapes=[pltpu.SMEM((1,), jnp.float32)],
    )
    out = pl.pallas_call(
        _gcpl_kernel,
        grid_spec=grid_spec,
        out_shape=jax.ShapeDtypeStruct((1, 1), jnp.float32),
        compiler_params=pltpu.CompilerParams(
            dimension_semantics=("arbitrary",)),
    )(label_arr, prototypes, prototypes, sample_counts.T, feature)
    return out[0, 0]
